# baseline (speedup 1.0000x reference)
import functools

import jax
import jax.numpy as jnp
from jax import lax
from jax.experimental import pallas as pl
from jax.experimental.pallas import tpu as pltpu
from jax.experimental.pallas import tpu_sc as plsc

NC, NS = 2, 16
EB = 128
LAT = 256


def _ln(h, g, be):
    mu = jnp.mean(h, axis=-1, keepdims=True)
    d = h - mu
    var = jnp.mean(d * d, axis=-1, keepdims=True)
    return d * lax.rsqrt(var + 1e-5) * g + be


def _mlp2_body(n_mm, n_add, *refs):
    xs = refs[:n_mm]
    ws = refs[n_mm:2 * n_mm]
    adds = refs[2 * n_mm:2 * n_mm + n_add]
    b1, g1, be1, w2, b2, g2, be2, out = refs[2 * n_mm + n_add:]
    acc = jnp.dot(xs[0][...], ws[0][...], preferred_element_type=jnp.float32)
    for k in range(1, n_mm):
        acc = acc + jnp.dot(xs[k][...], ws[k][...],
                            preferred_element_type=jnp.float32)
    for a in adds:
        acc = acc + a[...]
    h = _ln(jax.nn.relu(acc + b1[...]), g1[...], be1[...])
    h2 = jax.nn.relu(jnp.dot(h, w2[...], preferred_element_type=jnp.float32)
                     + b2[...])
    out[...] = _ln(h2, g2[...], be2[...])


def _mlp2(mm_inputs, add_inputs, l1, l2, m, block_rows):
    n_mm, n_add = len(mm_inputs), len(add_inputs)
    b1, g1, be1 = (p.reshape(1, LAT) for p in l1)
    w2 = l2[0]
    b2, g2, be2 = (p.reshape(1, LAT) for p in l2[1:])
    row_spec = lambda k: pl.BlockSpec((block_rows, k), lambda i: (i, 0))
    full_spec = lambda s: pl.BlockSpec(s, lambda i: (0, 0))
    in_specs = ([row_spec(x.shape[1]) for x, _ in mm_inputs]
                + [full_spec(w.shape) for _, w in mm_inputs]
                + [row_spec(LAT) for _ in add_inputs]
                + [full_spec((1, LAT))] * 3
                + [full_spec((LAT, LAT))]
                + [full_spec((1, LAT))] * 3)
    args = ([x for x, _ in mm_inputs] + [w for _, w in mm_inputs]
            + list(add_inputs) + [b1, g1, be1, w2, b2, g2, be2])
    return pl.pallas_call(
        functools.partial(_mlp2_body, n_mm, n_add),
        grid=(m // block_rows,),
        in_specs=in_specs,
        out_specs=pl.BlockSpec((block_rows, LAT), lambda i: (i, 0)),
        out_shape=jax.ShapeDtypeStruct((m, LAT), jnp.float32),
    )(*args)


def _proj_body(x_ref, wa_ref, wb_ref, oa_ref, ob_ref):
    x = x_ref[...]
    oa_ref[...] = jnp.dot(x, wa_ref[...], preferred_element_type=jnp.float32)
    ob_ref[...] = jnp.dot(x, wb_ref[...], preferred_element_type=jnp.float32)


def _proj2(x, wa, wb, block_rows):
    m = x.shape[0]
    return pl.pallas_call(
        _proj_body,
        grid=(m // block_rows,),
        in_specs=[pl.BlockSpec((block_rows, LAT), lambda i: (i, 0)),
                  pl.BlockSpec((LAT, LAT), lambda i: (0, 0)),
                  pl.BlockSpec((LAT, LAT), lambda i: (0, 0))],
        out_specs=[pl.BlockSpec((block_rows, LAT), lambda i: (i, 0)),
                   pl.BlockSpec((block_rows, LAT), lambda i: (i, 0))],
        out_shape=[jax.ShapeDtypeStruct((m, LAT), jnp.float32),
                   jax.ShapeDtypeStruct((m, LAT), jnp.float32)],
    )(x, wa, wb)


def _sc_gather2(ta, ia2, tb, ib2):
    nb = ia2.shape[0]
    e, d = nb * EB, ta.shape[1]
    nw = NC * NS
    nbw = nb // nw
    mesh = plsc.VectorSubcoreMesh(core_axis_name="c", subcore_axis_name="s")

    @functools.partial(
        pl.kernel, mesh=mesh,
        out_type=[jax.ShapeDtypeStruct((e, d), jnp.float32),
                  jax.ShapeDtypeStruct((e, d), jnp.float32)],
        scratch_types=[pltpu.VMEM((nbw, EB), jnp.int32),
                       pltpu.VMEM((EB, d), jnp.float32),
                       pltpu.VMEM((EB, d), jnp.float32),
                       pltpu.SemaphoreType.DMA, pltpu.SemaphoreType.DMA,
                       pltpu.SemaphoreType.DMA, pltpu.SemaphoreType.DMA],
    )
    def k(ta_h, ia_h, tb_h, ib_h, oa_h, ob_h,
          idx_v, r0, r1, sg0, sg1, so0, so1):
        wid = lax.axis_index("s") * NC + lax.axis_index("c")
        rows = (r0, r1)
        sg = (sg0, sg1)
        so = (so0, so1)

        def run(t_h, i2_h, o_h):
            pltpu.sync_copy(i2_h.at[pl.ds(wid * nbw, nbw)], idx_v)

            def start_g(b, p):
                pltpu.async_copy(t_h.at[idx_v.at[b]], rows[p], sg[p])

            def wait_g(p):
                pltpu.make_async_copy(t_h.at[idx_v.at[0]], rows[p],
                                      sg[p]).wait()

            def start_s(b, p):
                off = (wid * nbw + b) * EB
                pltpu.async_copy(rows[p], o_h.at[pl.ds(off, EB)], so[p])

            def wait_s(p):
                pltpu.make_async_copy(rows[p],
                                      o_h.at[pl.ds(0, EB)], so[p]).wait()

            start_g(0, 0)
            start_g(1, 1)

            def pair(k2, carry):
                b0 = 2 * k2
                b1 = b0 + 1
                wait_g(0)
                start_s(b0, 0)
                wait_g(1)
                start_s(b1, 1)

                @pl.when(b0 + 2 < nbw)
                def _():
                    wait_s(0)
                    start_g(b0 + 2, 0)
                    wait_s(1)
                    start_g(b1 + 2, 1)

                return carry

            lax.fori_loop(0, nbw // 2, pair, 0)
            wait_s(0)
            wait_s(1)

        run(ta_h, ia_h, oa_h)
        run(tb_h, ib_h, ob_h)

    return k(ta, ia2, tb, ib2)


def _sc_segsum(e_arr, recv2, zeros, n):
    nb = recv2.shape[0]
    half = LAT // NC
    n_pad = ((n + NS * 8 - 1) // (NS * 8)) * (NS * 8)
    rows_per_tile = n_pad // NS
    nbs = nb // NS
    mesh = plsc.VectorSubcoreMesh(core_axis_name="c", subcore_axis_name="s")

    @functools.partial(
        pl.kernel, mesh=mesh,
        out_type=jax.ShapeDtypeStruct((NC, n_pad, half), jnp.float32),
        scratch_types=[pltpu.VMEM((nbs, EB), jnp.int32),
                       pltpu.VMEM((EB, half), jnp.float32),
                       pltpu.VMEM((EB, half), jnp.float32),
                       pltpu.VMEM_SHARED((n_pad, half), jnp.float32),
                       pltpu.SemaphoreType.DMA, pltpu.SemaphoreType.DMA],
    )
    def k(e_hbm, r_hbm, z_hbm, out_hbm, idxs, eb0, eb1, shared, se0, se1):
        c = lax.axis_index("c")
        s = lax.axis_index("s")
        ebuf = (eb0, eb1)
        se = (se0, se1)
        pltpu.sync_copy(z_hbm, shared.at[pl.ds(s * rows_per_tile, rows_per_tile)])
        pltpu.sync_copy(r_hbm.at[pl.ds(s * nbs, nbs)], idxs)
        plsc.subcore_barrier()

        def start_e(b, p):
            base = (s * nbs + b) * EB
            pltpu.async_copy(
                e_hbm.at[pl.ds(base, EB), pl.ds(c * half, half)],
                ebuf[p], se[p])

        def wait_e(p):
            pltpu.make_async_copy(
                e_hbm.at[pl.ds(0, EB), pl.ds(0, half)], ebuf[p],
                se[p]).wait()

        start_e(0, 0)
        start_e(1, 1)

        def pair(k2, carry):
            b0 = 2 * k2
            b1 = b0 + 1
            wait_e(0)
            pltpu.sync_copy(ebuf[0], shared.at[idxs.at[b0]], add=True)

            @pl.when(b0 + 2 < nbs)
            def _():
                start_e(b0 + 2, 0)

            wait_e(1)
            pltpu.sync_copy(ebuf[1], shared.at[idxs.at[b1]], add=True)

            @pl.when(b1 + 2 < nbs)
            def _():
                start_e(b1 + 2, 1)

            return carry

        lax.fori_loop(0, nbs // 2, pair, 0)
        plsc.subcore_barrier()
        pltpu.sync_copy(shared.at[pl.ds(s * rows_per_tile, rows_per_tile)],
                        out_hbm.at[c, pl.ds(s * rows_per_tile, rows_per_tile)])

    return k(e_arr, recv2, zeros)


def kernel(nodes, edges, senders, receivers, n_node, params):
    n, f = nodes.shape
    e_cnt = edges.shape[0]
    nw = NC * NS
    quantum = EB * nw * 2
    e_pad = ((e_cnt + quantum - 1) // quantum) * quantum
    pad = e_pad - e_cnt
    n_pad = ((n + NS * 8 - 1) // (NS * 8)) * (NS * 8)

    send_g = jnp.concatenate(
        [senders, jnp.zeros((pad,), jnp.int32)]).reshape(-1, EB)
    recv_g = jnp.concatenate(
        [receivers, jnp.zeros((pad,), jnp.int32)]).reshape(-1, EB)
    recv_s = jnp.concatenate(
        [receivers, jnp.full((pad,), n, jnp.int32)]).reshape(-1, EB)

    sp = params['sender']
    x = _mlp2([(nodes, sp[0][0])], [], sp[0][1:], sp[1], m=n, block_rows=1000)
    x = x + (jnp.asarray(n_node) - n).astype(jnp.float32)

    ep = params['edge0']
    k3 = ep[0][0].shape[0]
    edges8 = jnp.zeros((e_pad, 8), jnp.float32).at[:e_cnt, :k3].set(edges)
    w0 = jnp.concatenate([ep[0][0], jnp.zeros((8 - k3, LAT), jnp.float32)],
                         axis=0)
    e = _mlp2([(edges8, w0)], [], ep[0][1:], ep[1], m=e_pad, block_rows=640)

    zeros = jnp.zeros((n_pad // NS, LAT // NC), jnp.float32)

    for i in range(len(params['edge_steps'])):
        eps = params['edge_steps'][i]
        nps = params['node_steps'][i]
        w1 = eps[0][0]
        we, ws, wr = w1[:LAT], w1[LAT:2 * LAT], w1[2 * LAT:]
        ps, pr = _proj2(x, ws, wr, block_rows=1000)
        gs, gr = _sc_gather2(ps, send_g, pr, recv_g)
        e = _mlp2([(e, we)], [gs, gr], eps[0][1:], eps[1],
                  m=e_pad, block_rows=640)
        agg = _sc_segsum(e, recv_s, zeros, n)
        wn1 = nps[0][0]
        half = LAT // NC
        x = _mlp2([(x, wn1[:LAT]),
                   (agg[0], wn1[LAT:LAT + half]),
                   (agg[1], wn1[LAT + half:])],
                  [], nps[0][1:], nps[1], m=n, block_rows=1000)

    return (x, e[:e_cnt])


# speedup vs baseline: 1.0705x; 1.0705x over previous
import functools

import jax
import jax.numpy as jnp
from jax import lax
from jax.experimental import pallas as pl
from jax.experimental.pallas import tpu as pltpu
from jax.experimental.pallas import tpu_sc as plsc

NC, NS = 2, 16
EB = 128
LAT = 256


def _ln(h, g, be):
    mu = jnp.mean(h, axis=-1, keepdims=True)
    d = h - mu
    var = jnp.mean(d * d, axis=-1, keepdims=True)
    return d * lax.rsqrt(var + 1e-5) * g + be


def _mlp2_body(n_mm, n_add, *refs):
    xs = refs[:n_mm]
    ws = refs[n_mm:2 * n_mm]
    adds = refs[2 * n_mm:2 * n_mm + n_add]
    b1, g1, be1, w2, b2, g2, be2, out = refs[2 * n_mm + n_add:]
    acc = jnp.dot(xs[0][...], ws[0][...], preferred_element_type=jnp.float32)
    for k in range(1, n_mm):
        acc = acc + jnp.dot(xs[k][...], ws[k][...],
                            preferred_element_type=jnp.float32)
    for a in adds:
        acc = acc + a[...]
    h = _ln(jax.nn.relu(acc + b1[...]), g1[...], be1[...])
    h2 = jax.nn.relu(jnp.dot(h, w2[...], preferred_element_type=jnp.float32)
                     + b2[...])
    out[...] = _ln(h2, g2[...], be2[...])


def _mlp2(mm_inputs, add_inputs, l1, l2, m, block_rows):
    n_mm, n_add = len(mm_inputs), len(add_inputs)
    b1, g1, be1 = (p.reshape(1, LAT) for p in l1)
    w2 = l2[0]
    b2, g2, be2 = (p.reshape(1, LAT) for p in l2[1:])
    row_spec = lambda k: pl.BlockSpec((block_rows, k), lambda i: (i, 0))
    full_spec = lambda s: pl.BlockSpec(s, lambda i: (0, 0))
    in_specs = ([row_spec(x.shape[1]) for x, _ in mm_inputs]
                + [full_spec(w.shape) for _, w in mm_inputs]
                + [row_spec(LAT) for _ in add_inputs]
                + [full_spec((1, LAT))] * 3
                + [full_spec((LAT, LAT))]
                + [full_spec((1, LAT))] * 3)
    args = ([x for x, _ in mm_inputs] + [w for _, w in mm_inputs]
            + list(add_inputs) + [b1, g1, be1, w2, b2, g2, be2])
    return pl.pallas_call(
        functools.partial(_mlp2_body, n_mm, n_add),
        grid=(m // block_rows,),
        in_specs=in_specs,
        out_specs=pl.BlockSpec((block_rows, LAT), lambda i: (i, 0)),
        out_shape=jax.ShapeDtypeStruct((m, LAT), jnp.float32),
    )(*args)


def _proj_body(x_ref, wa_ref, wb_ref, oa_ref, ob_ref):
    x = x_ref[...]
    oa_ref[...] = jnp.dot(x, wa_ref[...], preferred_element_type=jnp.float32)
    ob_ref[...] = jnp.dot(x, wb_ref[...], preferred_element_type=jnp.float32)


def _proj2(x, wa, wb, block_rows):
    m = x.shape[0]
    return pl.pallas_call(
        _proj_body,
        grid=(m // block_rows,),
        in_specs=[pl.BlockSpec((block_rows, LAT), lambda i: (i, 0)),
                  pl.BlockSpec((LAT, LAT), lambda i: (0, 0)),
                  pl.BlockSpec((LAT, LAT), lambda i: (0, 0))],
        out_specs=[pl.BlockSpec((block_rows, LAT), lambda i: (i, 0)),
                   pl.BlockSpec((block_rows, LAT), lambda i: (i, 0))],
        out_shape=[jax.ShapeDtypeStruct((m, LAT), jnp.float32),
                   jax.ShapeDtypeStruct((m, LAT), jnp.float32)],
    )(x, wa, wb)


def _sc_gather2(ta, ia, tb, ib):
    e, d = ia.shape[0], ta.shape[1]
    nb = e // EB
    nw = NC * NS
    nbw = nb // nw
    mesh = plsc.VectorSubcoreMesh(core_axis_name="c", subcore_axis_name="s")

    @functools.partial(
        pl.kernel, mesh=mesh,
        out_type=[jax.ShapeDtypeStruct((e, d), jnp.float32),
                  jax.ShapeDtypeStruct((e, d), jnp.float32)],
        scratch_types=[pltpu.VMEM((EB,), jnp.int32),
                       pltpu.VMEM((EB,), jnp.int32),
                       pltpu.VMEM((EB, d), jnp.float32),
                       pltpu.VMEM((EB, d), jnp.float32),
                       pltpu.SemaphoreType.DMA, pltpu.SemaphoreType.DMA,
                       pltpu.SemaphoreType.DMA, pltpu.SemaphoreType.DMA,
                       pltpu.SemaphoreType.DMA, pltpu.SemaphoreType.DMA],
    )
    def k(ta_h, ia_h, tb_h, ib_h, oa_h, ob_h,
          i0, i1, r0, r1, si0, si1, sg0, sg1, so0, so1):
        wid = lax.axis_index("s") * NC + lax.axis_index("c")
        ibuf = (i0, i1)
        rows = (r0, r1)
        si = (si0, si1)
        sg = (sg0, sg1)
        so = (so0, so1)

        def run(t_h, i_h, o_h):
            def off(b):
                return (b * nw + wid) * EB

            def start_i(b, p):
                pltpu.async_copy(i_h.at[pl.ds(off(b), EB)], ibuf[p], si[p])

            def wait_i(p):
                pltpu.make_async_copy(i_h.at[pl.ds(0, EB)], ibuf[p],
                                      si[p]).wait()

            def start_g(p):
                pltpu.async_copy(t_h.at[ibuf[p]], rows[p], sg[p])

            def wait_g(p):
                pltpu.make_async_copy(t_h.at[ibuf[p]], rows[p], sg[p]).wait()

            def start_s(b, p):
                pltpu.async_copy(rows[p], o_h.at[pl.ds(off(b), EB)], so[p])

            def wait_s(p):
                pltpu.make_async_copy(rows[p],
                                      o_h.at[pl.ds(0, EB)], so[p]).wait()

            start_i(0, 0)
            start_i(1, 1)

            def pair(k2, carry):
                b0 = 2 * k2
                b1 = b0 + 1

                @pl.when(k2 > 0)
                def _():
                    wait_s(0)
                    wait_s(1)

                wait_i(0)
                start_g(0)
                wait_i(1)
                start_g(1)
                wait_g(0)

                @pl.when(b0 + 2 < nbw)
                def _():
                    start_i(b0 + 2, 0)

                start_s(b0, 0)
                wait_g(1)

                @pl.when(b1 + 2 < nbw)
                def _():
                    start_i(b1 + 2, 1)

                start_s(b1, 1)
                return carry

            lax.fori_loop(0, nbw // 2, pair, 0)
            wait_s(0)
            wait_s(1)

        run(ta_h, ia_h, oa_h)
        run(tb_h, ib_h, ob_h)

    return k(ta, ia, tb, ib)


def _sc_segsum(e_arr, recv2, zeros, n):
    nb = recv2.shape[0]
    half = LAT // NC
    n_pad = ((n + NS * 8 - 1) // (NS * 8)) * (NS * 8)
    rows_per_tile = n_pad // NS
    nbs = nb // NS
    mesh = plsc.VectorSubcoreMesh(core_axis_name="c", subcore_axis_name="s")

    @functools.partial(
        pl.kernel, mesh=mesh,
        out_type=jax.ShapeDtypeStruct((NC, n_pad, half), jnp.float32),
        scratch_types=[pltpu.VMEM((nbs, EB), jnp.int32),
                       pltpu.VMEM((EB, half), jnp.float32),
                       pltpu.VMEM((EB, half), jnp.float32),
                       pltpu.VMEM_SHARED((n_pad, half), jnp.float32),
                       pltpu.SemaphoreType.DMA, pltpu.SemaphoreType.DMA],
    )
    def k(e_hbm, r_hbm, z_hbm, out_hbm, idxs, eb0, eb1, shared, se0, se1):
        c = lax.axis_index("c")
        s = lax.axis_index("s")
        ebuf = (eb0, eb1)
        se = (se0, se1)
        pltpu.sync_copy(z_hbm, shared.at[pl.ds(s * rows_per_tile, rows_per_tile)])
        pltpu.sync_copy(r_hbm.at[pl.ds(s * nbs, nbs)], idxs)
        plsc.subcore_barrier()

        def start_e(b, p):
            base = (s * nbs + b) * EB
            pltpu.async_copy(
                e_hbm.at[pl.ds(base, EB), pl.ds(c * half, half)],
                ebuf[p], se[p])

        def wait_e(p):
            pltpu.make_async_copy(
                e_hbm.at[pl.ds(0, EB), pl.ds(0, half)], ebuf[p],
                se[p]).wait()

        start_e(0, 0)
        start_e(1, 1)

        def pair(k2, carry):
            b0 = 2 * k2
            b1 = b0 + 1
            wait_e(0)
            pltpu.sync_copy(ebuf[0], shared.at[idxs.at[b0]], add=True)

            @pl.when(b0 + 2 < nbs)
            def _():
                start_e(b0 + 2, 0)

            wait_e(1)
            pltpu.sync_copy(ebuf[1], shared.at[idxs.at[b1]], add=True)

            @pl.when(b1 + 2 < nbs)
            def _():
                start_e(b1 + 2, 1)

            return carry

        lax.fori_loop(0, nbs // 2, pair, 0)
        plsc.subcore_barrier()
        pltpu.sync_copy(shared.at[pl.ds(s * rows_per_tile, rows_per_tile)],
                        out_hbm.at[c, pl.ds(s * rows_per_tile, rows_per_tile)])

    return k(e_arr, recv2, zeros)


def kernel(nodes, edges, senders, receivers, n_node, params):
    n, f = nodes.shape
    e_cnt = edges.shape[0]
    nw = NC * NS
    quantum = EB * nw * 2
    e_pad = ((e_cnt + quantum - 1) // quantum) * quantum
    pad = e_pad - e_cnt
    n_pad = ((n + NS * 8 - 1) // (NS * 8)) * (NS * 8)

    send_g = jnp.concatenate([senders, jnp.zeros((pad,), jnp.int32)])
    recv_g = jnp.concatenate([receivers, jnp.zeros((pad,), jnp.int32)])
    recv_s = jnp.concatenate(
        [receivers, jnp.full((pad,), n, jnp.int32)]).reshape(-1, EB)

    sp = params['sender']
    x = _mlp2([(nodes, sp[0][0])], [], sp[0][1:], sp[1], m=n, block_rows=1000)
    x = x + (jnp.asarray(n_node) - n).astype(jnp.float32)

    ep = params['edge0']
    k3 = ep[0][0].shape[0]
    edges8 = jnp.zeros((e_pad, 8), jnp.float32).at[:e_cnt, :k3].set(edges)
    w0 = jnp.concatenate([ep[0][0], jnp.zeros((8 - k3, LAT), jnp.float32)],
                         axis=0)
    e = _mlp2([(edges8, w0)], [], ep[0][1:], ep[1], m=e_pad, block_rows=640)

    zeros = jnp.zeros((n_pad // NS, LAT // NC), jnp.float32)

    for i in range(len(params['edge_steps'])):
        eps = params['edge_steps'][i]
        nps = params['node_steps'][i]
        w1 = eps[0][0]
        we, ws, wr = w1[:LAT], w1[LAT:2 * LAT], w1[2 * LAT:]
        ps, pr = _proj2(x, ws, wr, block_rows=1000)
        gs, gr = _sc_gather2(ps, send_g, pr, recv_g)
        e = _mlp2([(e, we)], [gs, gr], eps[0][1:], eps[1],
                  m=e_pad, block_rows=640)
        agg = _sc_segsum(e, recv_s, zeros, n)
        wn1 = nps[0][0]
        half = LAT // NC
        x = _mlp2([(x, wn1[:LAT]),
                   (agg[0], wn1[LAT:LAT + half]),
                   (agg[1], wn1[LAT + half:])],
                  [], nps[0][1:], nps[1], m=n, block_rows=1000)

    return (x, e[:e_cnt])
